# Initial kernel scaffold; baseline (speedup 1.0000x reference)
import functools

import jax
import jax.numpy as jnp
from jax import lax
from jax.experimental import pallas as pl
from jax.experimental.pallas import tpu as pltpu
from jax.experimental.pallas import tpu_sc as plsc

_N = 10000
_E = 320000
_D = 128
_NC = 2
_NS = 16
_NW = _NC * _NS
_EPT = _E // _NW
_K = 80
_NCHUNK = _EPT // _K
_RPT = _N // _NS

_mesh = plsc.VectorSubcoreMesh(core_axis_name="c", subcore_axis_name="s")


@functools.partial(
    pl.kernel,
    out_type=jax.ShapeDtypeStruct((_NC, _N, _D), jnp.float32),
    mesh=_mesh,
    scratch_types=[
        pltpu.VMEM((_K,), jnp.int32),
        pltpu.VMEM((_K,), jnp.int32),
        pltpu.VMEM((_K, _D), jnp.float32),
        pltpu.VMEM_SHARED((_N, _D), jnp.float32),
        pltpu.SemaphoreType.DMA,
    ],
)
def _sc_agg(x_hbm, src_hbm, dst_hbm, out_hbm, src_v, dst_v, rows_v, acc_sh, sem):
    c = lax.axis_index("c")
    s = lax.axis_index("s")
    wid = c * _NS + s
    r0 = s * _RPT
    pltpu.sync_copy(x_hbm.at[pl.ds(r0, _RPT)], acc_sh.at[pl.ds(r0, _RPT)])
    plsc.subcore_barrier()
    ebase = wid * _EPT

    @pl.loop(0, _NCHUNK)
    def _chunk(i):
        base = ebase + i * _K
        pltpu.sync_copy(src_hbm.at[pl.ds(base, _K)], src_v)
        pltpu.sync_copy(dst_hbm.at[pl.ds(base, _K)], dst_v)
        pltpu.async_copy(x_hbm.at[src_v], rows_v, sem).wait()
        pltpu.sync_copy(rows_v, acc_sh.at[dst_v], add=True)

    plsc.subcore_barrier()
    pltpu.sync_copy(acc_sh.at[pl.ds(r0, _RPT)], out_hbm.at[c, pl.ds(r0, _RPT)])


def _mlp_body(x_ref, agg_ref, w_ref, b_ref, out_ref):
    h = agg_ref[0] + agg_ref[1] - x_ref[...]
    out_ref[...] = (
        jnp.dot(h, w_ref[...], preferred_element_type=jnp.float32) + b_ref[...]
    )


_RB = 1000

_mlp = pl.pallas_call(
    _mlp_body,
    grid=(_N // _RB,),
    in_specs=[
        pl.BlockSpec((_RB, _D), lambda i: (i, 0)),
        pl.BlockSpec((_NC, _RB, _D), lambda i: (0, i, 0)),
        pl.BlockSpec((_D, _D), lambda i: (0, 0)),
        pl.BlockSpec((1, _D), lambda i: (0, 0)),
    ],
    out_specs=pl.BlockSpec((_RB, _D), lambda i: (i, 0)),
    out_shape=jax.ShapeDtypeStruct((_N, _D), jnp.float32),
)


def kernel(x, edge_index, W, b):
    src = edge_index[0]
    dst = edge_index[1]
    agg2 = _sc_agg(x, src, dst)
    return _mlp(x, agg2, W, b.reshape(1, _D))


# speedup vs baseline: 5.4900x; 5.4900x over previous
import functools

import jax
import jax.numpy as jnp
from jax import lax
from jax.experimental import pallas as pl
from jax.experimental.pallas import tpu as pltpu
from jax.experimental.pallas import tpu_sc as plsc

_N = 10000
_E = 320000
_D = 128
_NC = 2
_NS = 16
_NW = _NC * _NS
_EPT = _E // _NW
_K = 80
_NCHUNK = _EPT // _K
_RPT = 624
_RREM = _N - _NS * _RPT

_mesh = plsc.VectorSubcoreMesh(core_axis_name="c", subcore_axis_name="s")


@functools.partial(
    pl.kernel,
    out_type=jax.ShapeDtypeStruct((_NC, _N, _D), jnp.float32),
    mesh=_mesh,
    scratch_types=[
        pltpu.VMEM((_K,), jnp.int32),
        pltpu.VMEM((_K,), jnp.int32),
        pltpu.VMEM((_K, _D), jnp.float32),
        pltpu.VMEM_SHARED((_N, _D), jnp.float32),
        pltpu.SemaphoreType.DMA,
    ],
)
def _sc_agg(x_hbm, src_hbm, dst_hbm, out_hbm, src_v, dst_v, rows_v, acc_sh, sem):
    c = lax.axis_index("c")
    s = lax.axis_index("s")
    wid = c * _NS + s
    r0 = s * _RPT
    pltpu.sync_copy(x_hbm.at[pl.ds(r0, _RPT)], acc_sh.at[pl.ds(r0, _RPT)])

    @pl.when(s == _NS - 1)
    def _init_tail():
        rt = _NS * _RPT
        pltpu.sync_copy(x_hbm.at[pl.ds(rt, _RREM)], acc_sh.at[pl.ds(rt, _RREM)])

    plsc.subcore_barrier()
    ebase = wid * _EPT

    @pl.loop(0, _NCHUNK)
    def _chunk(i):
        base = ebase + i * _K
        pltpu.sync_copy(src_hbm.at[pl.ds(base, _K)], src_v)
        pltpu.sync_copy(dst_hbm.at[pl.ds(base, _K)], dst_v)
        pltpu.async_copy(x_hbm.at[src_v], rows_v, sem).wait()
        pltpu.sync_copy(rows_v, acc_sh.at[dst_v], add=True)

    plsc.subcore_barrier()
    pltpu.sync_copy(acc_sh.at[pl.ds(r0, _RPT)], out_hbm.at[c, pl.ds(r0, _RPT)])

    @pl.when(s == _NS - 1)
    def _wb_tail():
        rt = _NS * _RPT
        pltpu.sync_copy(acc_sh.at[pl.ds(rt, _RREM)], out_hbm.at[c, pl.ds(rt, _RREM)])


def _mlp_body(x_ref, agg_ref, w_ref, b_ref, out_ref):
    h = agg_ref[0] + agg_ref[1] - x_ref[...]
    out_ref[...] = (
        jnp.dot(h, w_ref[...], preferred_element_type=jnp.float32) + b_ref[...]
    )


_RB = 1000

_mlp = pl.pallas_call(
    _mlp_body,
    grid=(_N // _RB,),
    in_specs=[
        pl.BlockSpec((_RB, _D), lambda i: (i, 0)),
        pl.BlockSpec((_NC, _RB, _D), lambda i: (0, i, 0)),
        pl.BlockSpec((_D, _D), lambda i: (0, 0)),
        pl.BlockSpec((1, _D), lambda i: (0, 0)),
    ],
    out_specs=pl.BlockSpec((_RB, _D), lambda i: (i, 0)),
    out_shape=jax.ShapeDtypeStruct((_N, _D), jnp.float32),
)


def kernel(x, edge_index, W, b):
    src = edge_index[0]
    dst = edge_index[1]
    agg2 = _sc_agg(x, src, dst)
    return _mlp(x, agg2, W, b.reshape(1, _D))


# speedup vs baseline: 12.5776x; 2.2910x over previous
import functools

import jax
import jax.numpy as jnp
from jax import lax
from jax.experimental import pallas as pl
from jax.experimental.pallas import tpu as pltpu
from jax.experimental.pallas import tpu_sc as plsc

_N = 10000
_E = 320000
_D = 128
_NC = 2
_NS = 16
_NW = _NC * _NS
_EPT = _E // _NW
_K = 125
_NCHUNK = _EPT // _K
_RPT = 624
_RREM = _N - _NS * _RPT

_mesh = plsc.VectorSubcoreMesh(core_axis_name="c", subcore_axis_name="s")


@functools.partial(
    pl.kernel,
    out_type=jax.ShapeDtypeStruct((_NC, _N, _D), jnp.float32),
    mesh=_mesh,
    scratch_types=[
        pltpu.VMEM((_NCHUNK, _K), jnp.int32),
        pltpu.VMEM((_K,), jnp.int32),
        pltpu.VMEM((_K,), jnp.int32),
        pltpu.VMEM((_K, _D), jnp.float32),
        pltpu.VMEM((_K, _D), jnp.float32),
        pltpu.VMEM_SHARED((_N, _D), jnp.float32),
        pltpu.SemaphoreType.DMA,
        pltpu.SemaphoreType.DMA,
        pltpu.SemaphoreType.DMA,
        pltpu.SemaphoreType.DMA,
    ],
)
def _sc_agg(x_hbm, src_hbm, dst_hbm, out_hbm,
            src_v, dst0_v, dst1_v, rows0_v, rows1_v, acc_sh,
            gsem0, gsem1, dsem0, dsem1):
    c = lax.axis_index("c")
    s = lax.axis_index("s")
    wid = c * _NS + s
    r0 = s * _RPT
    pltpu.sync_copy(x_hbm.at[pl.ds(r0, _RPT)], acc_sh.at[pl.ds(r0, _RPT)])

    @pl.when(s == _NS - 1)
    def _init_tail():
        rt = _NS * _RPT
        pltpu.sync_copy(x_hbm.at[pl.ds(rt, _RREM)], acc_sh.at[pl.ds(rt, _RREM)])

    plsc.subcore_barrier()

    pltpu.sync_copy(src_hbm.at[wid], src_v)

    rows = (rows0_v, rows1_v)
    dsts = (dst0_v, dst1_v)
    gsems = (gsem0, gsem1)
    dsems = (dsem0, dsem1)
    for b in range(2):
        pltpu.async_copy(dst_hbm.at[wid, b], dsts[b], dsems[b])
        pltpu.async_copy(x_hbm.at[src_v.at[b]], rows[b], gsems[b])

    @pl.loop(0, _NCHUNK, step=2)
    def _chunk(g):
        for b in range(2):
            i = g + b
            pltpu.make_async_copy(dst_hbm.at[wid, i], dsts[b], dsems[b]).wait()
            pltpu.make_async_copy(x_hbm.at[src_v.at[i]], rows[b], gsems[b]).wait()
            pltpu.sync_copy(rows[b], acc_sh.at[dsts[b]], add=True)

            @pl.when(i + 2 < _NCHUNK)
            def _next():
                pltpu.async_copy(dst_hbm.at[wid, i + 2], dsts[b], dsems[b])
                pltpu.async_copy(x_hbm.at[src_v.at[i + 2]], rows[b], gsems[b])

    plsc.subcore_barrier()
    pltpu.sync_copy(acc_sh.at[pl.ds(r0, _RPT)], out_hbm.at[c, pl.ds(r0, _RPT)])

    @pl.when(s == _NS - 1)
    def _wb_tail():
        rt = _NS * _RPT
        pltpu.sync_copy(acc_sh.at[pl.ds(rt, _RREM)], out_hbm.at[c, pl.ds(rt, _RREM)])


def _mlp_body(x_ref, agg_ref, w_ref, b_ref, out_ref):
    h = agg_ref[0] + agg_ref[1] - x_ref[...]
    out_ref[...] = (
        jnp.dot(h, w_ref[...], preferred_element_type=jnp.float32) + b_ref[...]
    )


_RB = 1000

_mlp = pl.pallas_call(
    _mlp_body,
    grid=(_N // _RB,),
    in_specs=[
        pl.BlockSpec((_RB, _D), lambda i: (i, 0)),
        pl.BlockSpec((_NC, _RB, _D), lambda i: (0, i, 0)),
        pl.BlockSpec((_D, _D), lambda i: (0, 0)),
        pl.BlockSpec((1, _D), lambda i: (0, 0)),
    ],
    out_specs=pl.BlockSpec((_RB, _D), lambda i: (i, 0)),
    out_shape=jax.ShapeDtypeStruct((_N, _D), jnp.float32),
)


def kernel(x, edge_index, W, b):
    src = edge_index[0].reshape(_NW, _NCHUNK, _K)
    dst = edge_index[1].reshape(_NW, _NCHUNK, _K)
    agg2 = _sc_agg(x, src, dst)
    return _mlp(x, agg2, W, b.reshape(1, _D))


# speedup vs baseline: 13.2200x; 1.0511x over previous
import functools

import jax
import jax.numpy as jnp
from jax import lax
from jax.experimental import pallas as pl
from jax.experimental.pallas import tpu as pltpu
from jax.experimental.pallas import tpu_sc as plsc

_N = 10000
_E = 320000
_D = 128
_NC = 2
_NS = 16
_NW = _NC * _NS
_EPT = _E // _NW
_K = 80
_NCHUNK = _EPT // _K
_NBUF = 3
_RPT = 624
_RREM = _N - _NS * _RPT

_mesh = plsc.VectorSubcoreMesh(core_axis_name="c", subcore_axis_name="s")


@functools.partial(
    pl.kernel,
    out_type=jax.ShapeDtypeStruct((_NC, _N, _D), jnp.float32),
    mesh=_mesh,
    scratch_types=[
        pltpu.VMEM((_NCHUNK, _K), jnp.int32),
        pltpu.VMEM((1, _K), jnp.int32),
        pltpu.VMEM((1, _K), jnp.int32),
        pltpu.VMEM((1, _K), jnp.int32),
        pltpu.VMEM((_K, _D), jnp.float32),
        pltpu.VMEM((_K, _D), jnp.float32),
        pltpu.VMEM((_K, _D), jnp.float32),
        pltpu.VMEM_SHARED((_N, _D), jnp.float32),
        pltpu.SemaphoreType.DMA,
        pltpu.SemaphoreType.DMA,
        pltpu.SemaphoreType.DMA,
        pltpu.SemaphoreType.DMA,
        pltpu.SemaphoreType.DMA,
        pltpu.SemaphoreType.DMA,
        pltpu.SemaphoreType.DMA,
        pltpu.SemaphoreType.DMA,
        pltpu.SemaphoreType.DMA,
    ],
)
def _sc_agg(x_hbm, src_hbm, dst_hbm, out_hbm,
            src_v, dst0_v, dst1_v, dst2_v, rows0_v, rows1_v, rows2_v, acc_sh,
            gsem0, gsem1, gsem2, dsem0, dsem1, dsem2, ssem0, ssem1, ssem2):
    c = lax.axis_index("c")
    s = lax.axis_index("s")
    wid = c * _NS + s
    r0 = s * _RPT
    pltpu.sync_copy(x_hbm.at[pl.ds(r0, _RPT)], acc_sh.at[pl.ds(r0, _RPT)])

    @pl.when(s == _NS - 1)
    def _init_tail():
        rt = _NS * _RPT
        pltpu.sync_copy(x_hbm.at[pl.ds(rt, _RREM)], acc_sh.at[pl.ds(rt, _RREM)])

    plsc.subcore_barrier()

    pltpu.sync_copy(src_hbm.at[wid], src_v)

    gsems = (gsem0, gsem1, gsem2)
    dsems = (dsem0, dsem1, dsem2)
    ssems = (ssem0, ssem1, ssem2)
    dsts = (dst0_v, dst1_v, dst2_v)
    rows = (rows0_v, rows1_v, rows2_v)

    def _prefetch(i, b):
        pltpu.async_copy(dst_hbm.at[wid, pl.ds(i, 1)], dsts[b], dsems[b])
        pltpu.async_copy(x_hbm.at[src_v.at[i]], rows[b], gsems[b])

    def _drain_scatter(b):
        pltpu.make_async_copy(
            rows[b], acc_sh.at[dsts[b].at[0]], ssems[b]
        ).wait()

    def _step(i, b, drain_prev, prefetch_next):
        pltpu.make_async_copy(dst_hbm.at[wid, pl.ds(i, 1)], dsts[b], dsems[b]).wait()
        pltpu.make_async_copy(x_hbm.at[src_v.at[i]], rows[b], gsems[b]).wait()
        pltpu.async_copy(rows[b], acc_sh.at[dsts[b].at[0]], ssems[b], add=True)
        b2 = (b + 2) % _NBUF
        if drain_prev:
            _drain_scatter(b2)
        if prefetch_next:
            _prefetch(i + 2, b2)

    for b in range(2):
        _prefetch(b, b)
    _step(0, 0, drain_prev=False, prefetch_next=True)

    @pl.loop(1, _NCHUNK - 4, step=_NBUF)
    def _chunk(g):
        for k in range(_NBUF):
            _step(g + k, (1 + k) % _NBUF, drain_prev=True, prefetch_next=True)

    _step(_NCHUNK - 4, (_NCHUNK - 4) % _NBUF, drain_prev=True, prefetch_next=True)
    _step(_NCHUNK - 3, (_NCHUNK - 3) % _NBUF, drain_prev=True, prefetch_next=True)
    _step(_NCHUNK - 2, (_NCHUNK - 2) % _NBUF, drain_prev=True, prefetch_next=False)
    _step(_NCHUNK - 1, (_NCHUNK - 1) % _NBUF, drain_prev=True, prefetch_next=False)
    _drain_scatter((_NCHUNK - 1) % _NBUF)
    plsc.subcore_barrier()
    pltpu.sync_copy(acc_sh.at[pl.ds(r0, _RPT)], out_hbm.at[c, pl.ds(r0, _RPT)])

    @pl.when(s == _NS - 1)
    def _wb_tail():
        rt = _NS * _RPT
        pltpu.sync_copy(acc_sh.at[pl.ds(rt, _RREM)], out_hbm.at[c, pl.ds(rt, _RREM)])


def _mlp_body(x_ref, agg_ref, w_ref, b_ref, out_ref):
    h = agg_ref[0] + agg_ref[1] - x_ref[...]
    out_ref[...] = (
        jnp.dot(h, w_ref[...], preferred_element_type=jnp.float32) + b_ref[...]
    )


_RB = 1000

_mlp = pl.pallas_call(
    _mlp_body,
    grid=(_N // _RB,),
    in_specs=[
        pl.BlockSpec((_RB, _D), lambda i: (i, 0)),
        pl.BlockSpec((_NC, _RB, _D), lambda i: (0, i, 0)),
        pl.BlockSpec((_D, _D), lambda i: (0, 0)),
        pl.BlockSpec((1, _D), lambda i: (0, 0)),
    ],
    out_specs=pl.BlockSpec((_RB, _D), lambda i: (i, 0)),
    out_shape=jax.ShapeDtypeStruct((_N, _D), jnp.float32),
)


def kernel(x, edge_index, W, b):
    src = edge_index[0].reshape(_NW, _NCHUNK, _K)
    dst = edge_index[1].reshape(_NW, _NCHUNK, _K)
    agg2 = _sc_agg(x, src, dst)
    return _mlp(x, agg2, W, b.reshape(1, _D))


# speedup vs baseline: 14.2556x; 1.0783x over previous
import functools

import jax
import jax.numpy as jnp
from jax import lax
from jax.experimental import pallas as pl
from jax.experimental.pallas import tpu as pltpu
from jax.experimental.pallas import tpu_sc as plsc

_N = 10000
_E = 320000
_D = 128
_NC = 2
_NS = 16
_NW = _NC * _NS
_EPT = _E // _NW
_K = 125
_NCHUNK = _EPT // _K
_NBUF = 3
_RPT = 624
_RREM = _N - _NS * _RPT

_mesh = plsc.VectorSubcoreMesh(core_axis_name="c", subcore_axis_name="s")


@functools.partial(
    pl.kernel,
    out_type=jax.ShapeDtypeStruct((_NC, _N, _D), jnp.bfloat16),
    mesh=_mesh,
    compiler_params=pltpu.CompilerParams(use_tc_tiling_on_sc=False),
    scratch_types=[
        pltpu.VMEM((_NCHUNK, _K), jnp.int32),
        pltpu.VMEM((1, _K), jnp.int32),
        pltpu.VMEM((1, _K), jnp.int32),
        pltpu.VMEM((1, _K), jnp.int32),
        pltpu.VMEM((_K, _D), jnp.bfloat16),
        pltpu.VMEM((_K, _D), jnp.bfloat16),
        pltpu.VMEM((_K, _D), jnp.bfloat16),
        pltpu.VMEM_SHARED((_N, _D), jnp.bfloat16),
        pltpu.SemaphoreType.DMA,
        pltpu.SemaphoreType.DMA,
        pltpu.SemaphoreType.DMA,
        pltpu.SemaphoreType.DMA,
        pltpu.SemaphoreType.DMA,
        pltpu.SemaphoreType.DMA,
        pltpu.SemaphoreType.DMA,
        pltpu.SemaphoreType.DMA,
        pltpu.SemaphoreType.DMA,
    ],
)
def _sc_agg(x_hbm, src_hbm, dst_hbm, out_hbm,
            src_v, dst0_v, dst1_v, dst2_v, rows0_v, rows1_v, rows2_v, acc_sh,
            gsem0, gsem1, gsem2, dsem0, dsem1, dsem2, ssem0, ssem1, ssem2):
    c = lax.axis_index("c")
    s = lax.axis_index("s")
    wid = c * _NS + s
    r0 = s * _RPT
    pltpu.sync_copy(x_hbm.at[pl.ds(r0, _RPT)], acc_sh.at[pl.ds(r0, _RPT)])

    @pl.when(s == _NS - 1)
    def _init_tail():
        rt = _NS * _RPT
        pltpu.sync_copy(x_hbm.at[pl.ds(rt, _RREM)], acc_sh.at[pl.ds(rt, _RREM)])

    plsc.subcore_barrier()

    pltpu.sync_copy(src_hbm.at[wid], src_v)

    gsems = (gsem0, gsem1, gsem2)
    dsems = (dsem0, dsem1, dsem2)
    ssems = (ssem0, ssem1, ssem2)
    dsts = (dst0_v, dst1_v, dst2_v)
    rows = (rows0_v, rows1_v, rows2_v)

    def _prefetch(i, b):
        pltpu.async_copy(dst_hbm.at[wid, pl.ds(i, 1)], dsts[b], dsems[b])
        pltpu.async_copy(x_hbm.at[src_v.at[i]], rows[b], gsems[b])

    def _drain_scatter(b):
        pltpu.make_async_copy(
            rows[b], acc_sh.at[dsts[b].at[0]], ssems[b]
        ).wait()

    def _step(i, b, drain_prev, prefetch_next):
        pltpu.make_async_copy(dst_hbm.at[wid, pl.ds(i, 1)], dsts[b], dsems[b]).wait()
        pltpu.make_async_copy(x_hbm.at[src_v.at[i]], rows[b], gsems[b]).wait()
        pltpu.async_copy(rows[b], acc_sh.at[dsts[b].at[0]], ssems[b], add=True)
        b2 = (b + 2) % _NBUF
        if drain_prev:
            _drain_scatter(b2)
        if prefetch_next:
            _prefetch(i + 2, b2)

    for b in range(2):
        _prefetch(b, b)
    _step(0, 0, drain_prev=False, prefetch_next=True)

    @pl.loop(1, _NCHUNK - 4, step=_NBUF)
    def _chunk(g):
        for k in range(_NBUF):
            _step(g + k, (1 + k) % _NBUF, drain_prev=True, prefetch_next=True)

    for i in range(_NCHUNK - 4, _NCHUNK):
        _step(i, i % _NBUF, drain_prev=True, prefetch_next=(i + 2 < _NCHUNK))
    _drain_scatter((_NCHUNK - 1) % _NBUF)
    plsc.subcore_barrier()
    pltpu.sync_copy(acc_sh.at[pl.ds(r0, _RPT)], out_hbm.at[c, pl.ds(r0, _RPT)])

    @pl.when(s == _NS - 1)
    def _wb_tail():
        rt = _NS * _RPT
        pltpu.sync_copy(acc_sh.at[pl.ds(rt, _RREM)], out_hbm.at[c, pl.ds(rt, _RREM)])


def _mlp_body(x_ref, agg_ref, w_ref, b_ref, out_ref):
    agg = agg_ref[0].astype(jnp.float32) + agg_ref[1].astype(jnp.float32)
    h = agg - x_ref[...]
    out_ref[...] = (
        jnp.dot(h, w_ref[...], preferred_element_type=jnp.float32) + b_ref[...]
    )


_RB = 1000

_mlp = pl.pallas_call(
    _mlp_body,
    grid=(_N // _RB,),
    in_specs=[
        pl.BlockSpec((_RB, _D), lambda i: (i, 0)),
        pl.BlockSpec((_NC, _RB, _D), lambda i: (0, i, 0)),
        pl.BlockSpec((_D, _D), lambda i: (0, 0)),
        pl.BlockSpec((1, _D), lambda i: (0, 0)),
    ],
    out_specs=pl.BlockSpec((_RB, _D), lambda i: (i, 0)),
    out_shape=jax.ShapeDtypeStruct((_N, _D), jnp.float32),
)


def kernel(x, edge_index, W, b):
    src = edge_index[0].reshape(_NW, _NCHUNK, _K)
    dst = edge_index[1].reshape(_NW, _NCHUNK, _K)
    agg2 = _sc_agg(x.astype(jnp.bfloat16), src, dst)
    return _mlp(x, agg2, W, b.reshape(1, _D))
